# baseline (speedup 1.0000x reference)
import functools

import jax
import jax.numpy as jnp
from jax import lax
from jax.experimental import pallas as pl
from jax.experimental.pallas import tpu as pltpu
from jax.experimental.pallas import tpu_sc as plsc

N = 100000
E = 1600000
H = 32
HALF = N // 2


def _lrelu(v):
    return jnp.where(v > 0, v, 0.01 * v)


def _node_embed(x, Wi, bi):
    BN = 4000

    def body(x_ref, wi_ref, bi_ref, out_ref):
        out_ref[...] = (
            jnp.dot(x_ref[...], wi_ref[...], preferred_element_type=jnp.float32)
            + bi_ref[...]
        )

    return pl.pallas_call(
        body,
        grid=(N // BN,),
        in_specs=[
            pl.BlockSpec((BN, 3), lambda i: (i, 0)),
            pl.BlockSpec((3, H), lambda i: (0, 0)),
            pl.BlockSpec((1, H), lambda i: (0, 0)),
        ],
        out_specs=pl.BlockSpec((BN, H), lambda i: (i, 0)),
        out_shape=jax.ShapeDtypeStruct((N, H), jnp.float32),
    )(x, Wi, bi.reshape(1, H))


def _sc_gather(table, eidx):
    CH = 1024
    NCHUNK = (2 * E) // CH
    NW = 32
    PER_W = -(-NCHUNK // NW)
    mesh = plsc.VectorSubcoreMesh(core_axis_name="c", subcore_axis_name="s")

    @functools.partial(
        pl.kernel,
        mesh=mesh,
        compiler_params=pltpu.CompilerParams(use_tc_tiling_on_sc=False),
        out_type=jax.ShapeDtypeStruct((2 * E, H), jnp.float32),
        scratch_types=[
            pltpu.VMEM((CH,), jnp.int32),
            pltpu.VMEM((CH, H), jnp.float32),
            pltpu.SemaphoreType.DMA,
        ],
    )
    def k(tab, idx_hbm, out, idxv, rows, sem):
        w = lax.axis_index("s") * 2 + lax.axis_index("c")

        def body(i, carry):
            cid = w + NW * i

            @pl.when(cid < NCHUNK)
            def _():
                base = pl.multiple_of(cid * CH, CH)
                pltpu.sync_copy(idx_hbm.at[pl.ds(base, CH)], idxv)
                cps = [
                    pltpu.async_copy(
                        tab.at[idxv.at[pl.ds(s * 128, 128)]],
                        rows.at[pl.ds(s * 128, 128), :],
                        sem,
                    )
                    for s in range(CH // 128)
                ]
                for cp in cps:
                    cp.wait()
                pltpu.sync_copy(rows, out.at[pl.ds(base, CH), :])

            return carry

        lax.fori_loop(0, PER_W, body, 0)

    return k(table, eidx)


def _sc_scatter(h_new, dst):
    CH = 512
    NCHUNK = E // CH
    PER_T = -(-NCHUNK // 16)
    R = 50048
    ZCH = R // 128
    mesh = plsc.VectorSubcoreMesh(core_axis_name="c", subcore_axis_name="s")

    @functools.partial(
        pl.kernel,
        mesh=mesh,
        compiler_params=pltpu.CompilerParams(use_tc_tiling_on_sc=False),
        out_type=jax.ShapeDtypeStruct((N, H), jnp.float32),
        scratch_types=[
            pltpu.VMEM((CH,), jnp.int32),
            pltpu.VMEM((CH // 128, 128), jnp.int32),
            pltpu.VMEM((CH, H), jnp.float32),
            pltpu.VMEM_SHARED((R, H), jnp.float32),
            pltpu.SemaphoreType.DMA,
        ],
    )
    def k(hnew, dref, out, dbuf, ibuf, rbuf, table, sem):
        c = lax.axis_index("c")
        s = lax.axis_index("s")
        nbase = c * HALF
        zeros16 = jnp.zeros((16,), jnp.float32)

        def zrow(r, carry):
            rbuf[r, pl.ds(0, 16)] = zeros16
            rbuf[r, pl.ds(16, 16)] = zeros16
            return carry

        lax.fori_loop(0, 128, zrow, 0)

        def ztab(i, carry):
            z = s + 16 * i

            @pl.when(z < ZCH)
            def _():
                off = pl.multiple_of(z * 128, 128)
                pltpu.sync_copy(rbuf.at[pl.ds(0, 128), :], table.at[pl.ds(off, 128), :])

            return carry

        lax.fori_loop(0, -(-ZCH // 16), ztab, 0)
        plsc.subcore_barrier()

        def chunk(i, carry):
            cid = s + 16 * i

            @pl.when(cid < NCHUNK)
            def _():
                base = pl.multiple_of(cid * CH, CH)
                pltpu.sync_copy(dref.at[pl.ds(base, CH)], dbuf)
                pltpu.sync_copy(hnew.at[pl.ds(base, CH), :], rbuf)
                for j in range(CH // 16):
                    v = dbuf[pl.ds(j * 16, 16)]
                    loc = v - nbase
                    ok = (loc >= 0) & (loc < HALF)
                    loc = jnp.where(ok, loc, HALF)
                    ibuf[j // 8, pl.ds((j % 8) * 16, 16)] = loc
                for t in range(CH // 128):
                    pltpu.sync_copy(
                        rbuf.at[pl.ds(t * 128, 128), :],
                        table.at[ibuf.at[t]],
                        add=True,
                    )

            return carry

        lax.fori_loop(0, PER_T, chunk, 0)
        plsc.subcore_barrier()

        rows_per_t = HALF // 16
        roff = s * rows_per_t
        pltpu.sync_copy(
            table.at[pl.ds(roff, rows_per_t), :],
            out.at[pl.ds(nbase + roff, rows_per_t), :],
        )

    return k(h_new, dst)


def _edge_compute(gath2, hmsg, We, be, Wm1, bm1, Wm2, bm2,
                  Wn1, bn1, Wn2, bn2, Wd, bd):
    EB = 12800

    def body(g_ref, hm_ref, we_ref, be_ref, wm1_ref, bm1_ref,
             wm2_ref, bm2_ref, wn1_ref, bn1_ref, wn2_ref, bn2_ref,
             wd_ref, bd_ref, hnew_ref, y_ref):
        f32 = jnp.float32
        g = g_ref[...]
        hm = hm_ref[...]
        wm1 = wm1_ref[...]
        wa = wm1[0:H]
        wb = wm1[H:2 * H]
        wc = wm1[2 * H:3 * H]
        zer = jnp.zeros((H, H), f32)
        wem = jnp.dot(we_ref[...], wc, preferred_element_type=f32)
        bpre = jnp.dot(be_ref[...], wc, preferred_element_type=f32) + bm1_ref[...]
        wx = jnp.concatenate(
            [jnp.concatenate([wb, wn1_ref[...]], axis=1),
             jnp.concatenate([wa, zer], axis=1)], axis=0)
        wmm = jnp.concatenate([wem, zer], axis=1)
        b1 = jnp.concatenate([bpre, bn1_ref[...]], axis=1)
        p = _lrelu(
            jnp.dot(g, wx, preferred_element_type=f32)
            + jnp.dot(hm, wmm, preferred_element_type=f32)
            + b1
        )
        w2 = jnp.concatenate(
            [jnp.concatenate([wm2_ref[...], zer], axis=1),
             jnp.concatenate([zer, wn2_ref[...]], axis=1)], axis=0)
        b2 = jnp.concatenate([bm2_ref[...], bn2_ref[...]], axis=1)
        q = _lrelu(jnp.dot(p, w2, preferred_element_type=f32) + b2)
        hn = q[:, 0:H] + q[:, H:2 * H]
        hnew_ref[...] = hn
        z = jnp.dot(hn, wd_ref[...], preferred_element_type=f32) + bd_ref[...]
        z = z - jnp.max(z, axis=-1, keepdims=True)
        ez = jnp.exp(z)
        y_ref[...] = ez / jnp.sum(ez, axis=-1, keepdims=True)

    wspec = pl.BlockSpec((H, H), lambda i: (0, 0))
    bspec = pl.BlockSpec((1, H), lambda i: (0, 0))
    return pl.pallas_call(
        body,
        grid=(E // EB,),
        in_specs=[
            pl.BlockSpec((EB, 2 * H), lambda i: (i, 0)),
            pl.BlockSpec((EB, H), lambda i: (i, 0)),
            wspec, bspec,
            pl.BlockSpec((3 * H, H), lambda i: (0, 0)), bspec,
            wspec, bspec,
            wspec, bspec,
            wspec, bspec,
            pl.BlockSpec((H, 2), lambda i: (0, 0)),
            pl.BlockSpec((1, 2), lambda i: (0, 0)),
        ],
        out_specs=[
            pl.BlockSpec((EB, H), lambda i: (i, 0)),
            pl.BlockSpec((EB, 2), lambda i: (i, 0)),
        ],
        out_shape=[
            jax.ShapeDtypeStruct((E, H), jnp.float32),
            jax.ShapeDtypeStruct((E, 2), jnp.float32),
        ],
    )(gath2, hmsg, We, be.reshape(1, H), Wm1, bm1.reshape(1, H),
      Wm2, bm2.reshape(1, H), Wn1, bn1.reshape(1, H), Wn2, bn2.reshape(1, H),
      Wd, bd.reshape(1, 2))


def _beliefs(h2, a2, Wu, bu, Wb, bb):
    BN = 2000

    def body(h_ref, a_ref, wu_ref, bu_ref, wb_ref, bb_ref, out_ref):
        f32 = jnp.float32
        hn = h_ref[:, 1, :]
        ag = a_ref[:, 1, :]
        wu = wu_ref[...]
        z = (
            jnp.dot(hn, wu[0:H], preferred_element_type=f32)
            + jnp.dot(ag, wu[H:2 * H], preferred_element_type=f32)
            + bu_ref[...]
        )
        z = _lrelu(z)
        t = jnp.dot(z, wb_ref[...], preferred_element_type=f32) + bb_ref[...]
        t = t - jnp.max(t, axis=-1, keepdims=True)
        et = jnp.exp(t)
        out_ref[...] = et / jnp.sum(et, axis=-1, keepdims=True)

    return pl.pallas_call(
        body,
        grid=(HALF // BN,),
        in_specs=[
            pl.BlockSpec((BN, 2, H), lambda i: (i, 0, 0)),
            pl.BlockSpec((BN, 2, H), lambda i: (i, 0, 0)),
            pl.BlockSpec((2 * H, H), lambda i: (0, 0)),
            pl.BlockSpec((1, H), lambda i: (0, 0)),
            pl.BlockSpec((H, 2), lambda i: (0, 0)),
            pl.BlockSpec((1, 2), lambda i: (0, 0)),
        ],
        out_specs=pl.BlockSpec((BN, 2), lambda i: (i, 0)),
        out_shape=jax.ShapeDtypeStruct((HALF, 2), jnp.float32),
    )(h2, a2, Wu, bu.reshape(1, H), Wb, bb.reshape(1, 2))


def kernel(x, edge_index, h_msg, Wi, bi, We, be, Wm1, bm1, Wm2, bm2,
           Wn1, bn1, Wn2, bn2, Wu, bu, Wd, bd, Wb, bb):
    h_node = _node_embed(x, Wi, bi)

    eidx = edge_index.T.reshape(2 * E)
    gath = _sc_gather(h_node, eidx)
    gath2 = gath.reshape(E, 2 * H)

    h_new, y_msg = _edge_compute(gath2, h_msg, We, be, Wm1, bm1, Wm2, bm2,
                                 Wn1, bn1, Wn2, bn2, Wd, bd)

    aggr = _sc_scatter(h_new, edge_index[1])

    y_beliefs = _beliefs(h_node.reshape(HALF, 2, H), aggr.reshape(HALF, 2, H),
                         Wu, bu, Wb, bb)
    return (h_new, y_msg, y_beliefs)


# speedup vs baseline: 1.4349x; 1.4349x over previous
import functools

import jax
import jax.numpy as jnp
from jax import lax
from jax.experimental import pallas as pl
from jax.experimental.pallas import tpu as pltpu
from jax.experimental.pallas import tpu_sc as plsc

N = 100000
E = 1600000
H = 32
HALF = N // 2


def _lrelu(v):
    return jnp.where(v > 0, v, 0.01 * v)


def _node_embed(x, Wi, bi):
    BN = 4000

    def body(x_ref, wi_ref, bi_ref, out_ref):
        out_ref[...] = (
            jnp.dot(x_ref[...], wi_ref[...], preferred_element_type=jnp.float32)
            + bi_ref[...]
        )

    return pl.pallas_call(
        body,
        grid=(N // BN,),
        in_specs=[
            pl.BlockSpec((BN, 3), lambda i: (i, 0)),
            pl.BlockSpec((3, H), lambda i: (0, 0)),
            pl.BlockSpec((1, H), lambda i: (0, 0)),
        ],
        out_specs=pl.BlockSpec((BN, H), lambda i: (i, 0)),
        out_shape=jax.ShapeDtypeStruct((N, H), jnp.float32),
    )(x, Wi, bi.reshape(1, H))


def _sc_gather(table, src, dst):
    CH = 512
    NCHUNK = E // CH
    NW = 32
    PER_W = -(-NCHUNK // NW)
    mesh = plsc.VectorSubcoreMesh(core_axis_name="c", subcore_axis_name="s")

    @functools.partial(
        pl.kernel,
        mesh=mesh,
        compiler_params=pltpu.CompilerParams(use_tc_tiling_on_sc=False),
        out_type=[
            jax.ShapeDtypeStruct((E, H), jnp.float32),
            jax.ShapeDtypeStruct((E, H), jnp.float32),
        ],
        scratch_types=[
            pltpu.VMEM((CH,), jnp.int32),
            pltpu.VMEM((CH,), jnp.int32),
            pltpu.VMEM((CH, H), jnp.float32),
            pltpu.VMEM((CH, H), jnp.float32),
            pltpu.SemaphoreType.DMA,
        ],
    )
    def k(tab, src_hbm, dst_hbm, outs, outd, idxs, idxd, rs, rd, sem):
        w = lax.axis_index("s") * 2 + lax.axis_index("c")

        def body(i, carry):
            cid = w + NW * i

            @pl.when(cid < NCHUNK)
            def _():
                base = pl.multiple_of(cid * CH, CH)
                pltpu.sync_copy(src_hbm.at[pl.ds(base, CH)], idxs)
                pltpu.sync_copy(dst_hbm.at[pl.ds(base, CH)], idxd)
                cps = []
                for t in range(CH // 128):
                    cps.append(pltpu.async_copy(
                        tab.at[idxs.at[pl.ds(t * 128, 128)]],
                        rs.at[pl.ds(t * 128, 128), :], sem))
                    cps.append(pltpu.async_copy(
                        tab.at[idxd.at[pl.ds(t * 128, 128)]],
                        rd.at[pl.ds(t * 128, 128), :], sem))
                for cp in cps:
                    cp.wait()
                pltpu.sync_copy(rs, outs.at[pl.ds(base, CH), :])
                pltpu.sync_copy(rd, outd.at[pl.ds(base, CH), :])

            return carry

        lax.fori_loop(0, PER_W, body, 0)

    return k(table, src, dst)


def _sc_scatter(h_new, dst):
    CH = 512
    NCHUNK = E // CH
    PER_T = -(-NCHUNK // 16)
    R = 50048
    ZCH = R // 128
    mesh = plsc.VectorSubcoreMesh(core_axis_name="c", subcore_axis_name="s")

    @functools.partial(
        pl.kernel,
        mesh=mesh,
        compiler_params=pltpu.CompilerParams(use_tc_tiling_on_sc=False),
        out_type=jax.ShapeDtypeStruct((N, H), jnp.float32),
        scratch_types=[
            pltpu.VMEM((CH,), jnp.int32),
            pltpu.VMEM((CH // 128, 128), jnp.int32),
            pltpu.VMEM((CH, H), jnp.float32),
            pltpu.VMEM_SHARED((R, H), jnp.float32),
            pltpu.SemaphoreType.DMA,
        ],
    )
    def k(hnew, dref, out, dbuf, ibuf, rbuf, table, sem):
        c = lax.axis_index("c")
        s = lax.axis_index("s")
        nbase = c * HALF
        zeros16 = jnp.zeros((16,), jnp.float32)

        def zrow(r, carry):
            rbuf[r, pl.ds(0, 16)] = zeros16
            rbuf[r, pl.ds(16, 16)] = zeros16
            return carry

        lax.fori_loop(0, 128, zrow, 0)

        def ztab(i, carry):
            z = s + 16 * i

            @pl.when(z < ZCH)
            def _():
                off = pl.multiple_of(z * 128, 128)
                pltpu.sync_copy(rbuf.at[pl.ds(0, 128), :], table.at[pl.ds(off, 128), :])

            return carry

        lax.fori_loop(0, -(-ZCH // 16), ztab, 0)
        plsc.subcore_barrier()

        def chunk(i, carry):
            cid = s + 16 * i

            @pl.when(cid < NCHUNK)
            def _():
                base = pl.multiple_of(cid * CH, CH)
                pltpu.sync_copy(dref.at[pl.ds(base, CH)], dbuf)
                pltpu.sync_copy(hnew.at[pl.ds(base, CH), :], rbuf)
                for j in range(CH // 16):
                    v = dbuf[pl.ds(j * 16, 16)]
                    loc = v - nbase
                    ok = (loc >= 0) & (loc < HALF)
                    loc = jnp.where(ok, loc, HALF)
                    ibuf[j // 8, pl.ds((j % 8) * 16, 16)] = loc
                for t in range(CH // 128):
                    pltpu.sync_copy(
                        rbuf.at[pl.ds(t * 128, 128), :],
                        table.at[ibuf.at[t]],
                        add=True,
                    )

            return carry

        lax.fori_loop(0, PER_T, chunk, 0)
        plsc.subcore_barrier()

        rows_per_t = HALF // 16
        roff = s * rows_per_t
        pltpu.sync_copy(
            table.at[pl.ds(roff, rows_per_t), :],
            out.at[pl.ds(nbase + roff, rows_per_t), :],
        )

    return k(h_new, dst)


def _edge_compute(gs, gd, hmT, We, be, Wm1, bm1, Wm2, bm2,
                  Wn1, bn1, Wn2, bn2, Wd, bd):
    EB = 6400
    f32 = jnp.float32
    dn_lhsT = (((0,), (0,)), ((), ()))

    def body(gs_ref, gd_ref, hmT_ref, we_ref, be_ref, wm1_ref, bm1_ref,
             wm2_ref, bm2_ref, wn1_ref, bn1_ref, wn2_ref, bn2_ref,
             wd_ref, bd_ref, hnT_ref, hn_ref, y3_ref):
        wm1 = wm1_ref[...]
        wa = wm1[0:H]
        wb = wm1[H:2 * H]
        wc = wm1[2 * H:3 * H]
        zer = jnp.zeros((H, H), f32)
        wem = jnp.dot(we_ref[...], wc, preferred_element_type=f32)
        bpre = jnp.dot(be_ref[...], wc, preferred_element_type=f32) + bm1_ref[...]
        wa64 = jnp.concatenate([wa, zer], axis=1)
        wb64 = jnp.concatenate([wb, wn1_ref[...]], axis=1)
        wem64 = jnp.concatenate([wem, zer], axis=1)
        b1 = jnp.concatenate([bpre, bn1_ref[...]], axis=1)
        p = _lrelu(
            jnp.dot(gs_ref[...], wb64, preferred_element_type=f32)
            + jnp.dot(gd_ref[...], wa64, preferred_element_type=f32)
            + lax.dot_general(hmT_ref[...], wem64, dn_lhsT,
                              preferred_element_type=f32)
            + b1
        )
        w2 = jnp.concatenate(
            [jnp.concatenate([wm2_ref[...], zer], axis=1),
             jnp.concatenate([zer, wn2_ref[...]], axis=1)], axis=0)
        b2 = jnp.concatenate([bm2_ref[...], bn2_ref[...]], axis=1)
        q = _lrelu(jnp.dot(p, w2, preferred_element_type=f32) + b2)
        hn = q[:, 0:H] + q[:, H:2 * H]
        hn_ref[...] = hn
        qT = _lrelu(
            lax.dot_general(w2, p, (((0,), (1,)), ((), ())),
                            preferred_element_type=f32)
            + b2.reshape(2 * H, 1)
        )
        hnT = qT[0:H, :] + qT[H:2 * H, :]
        hnT_ref[...] = hnT
        zT = (lax.dot_general(wd_ref[...], hnT, dn_lhsT,
                              preferred_element_type=f32)
              + bd_ref[...].reshape(2, 1))
        zT = zT - jnp.max(zT, axis=0, keepdims=True)
        ez = jnp.exp(zT)
        yT = ez / jnp.sum(ez, axis=0, keepdims=True)
        y3_ref[...] = yT.reshape(2, EB // 128, 128).transpose(1, 0, 2)

    wspec = pl.BlockSpec((H, H), lambda i: (0, 0))
    bspec = pl.BlockSpec((1, H), lambda i: (0, 0))
    return pl.pallas_call(
        body,
        grid=(E // EB,),
        in_specs=[
            pl.BlockSpec((EB, H), lambda i: (i, 0)),
            pl.BlockSpec((EB, H), lambda i: (i, 0)),
            pl.BlockSpec((H, EB), lambda i: (0, i)),
            wspec, bspec,
            pl.BlockSpec((3 * H, H), lambda i: (0, 0)), bspec,
            wspec, bspec,
            wspec, bspec,
            wspec, bspec,
            pl.BlockSpec((H, 2), lambda i: (0, 0)),
            pl.BlockSpec((1, 2), lambda i: (0, 0)),
        ],
        out_specs=[
            pl.BlockSpec((H, EB), lambda i: (0, i)),
            pl.BlockSpec((EB, H), lambda i: (i, 0)),
            pl.BlockSpec((EB // 128, 2, 128), lambda i: (i, 0, 0)),
        ],
        out_shape=[
            jax.ShapeDtypeStruct((H, E), jnp.float32),
            jax.ShapeDtypeStruct((E, H), jnp.float32),
            jax.ShapeDtypeStruct((E // 128, 2, 128), jnp.float32),
        ],
    )(gs, gd, hmT, We, be.reshape(1, H), Wm1, bm1.reshape(1, H),
      Wm2, bm2.reshape(1, H), Wn1, bn1.reshape(1, H), Wn2, bn2.reshape(1, H),
      Wd, bd.reshape(1, 2))


def _beliefs(h2, a2, Wu, bu, Wb, bb):
    BN = 2000

    def body(h_ref, a_ref, wu_ref, bu_ref, wb_ref, bb_ref, out_ref):
        f32 = jnp.float32
        hn = h_ref[:, 1, :]
        ag = a_ref[:, 1, :]
        wu = wu_ref[...]
        z = (
            jnp.dot(hn, wu[0:H], preferred_element_type=f32)
            + jnp.dot(ag, wu[H:2 * H], preferred_element_type=f32)
            + bu_ref[...]
        )
        z = _lrelu(z)
        t = jnp.dot(z, wb_ref[...], preferred_element_type=f32) + bb_ref[...]
        t = t - jnp.max(t, axis=-1, keepdims=True)
        et = jnp.exp(t)
        out_ref[...] = et / jnp.sum(et, axis=-1, keepdims=True)

    return pl.pallas_call(
        body,
        grid=(HALF // BN,),
        in_specs=[
            pl.BlockSpec((BN, 2, H), lambda i: (i, 0, 0)),
            pl.BlockSpec((BN, 2, H), lambda i: (i, 0, 0)),
            pl.BlockSpec((2 * H, H), lambda i: (0, 0)),
            pl.BlockSpec((1, H), lambda i: (0, 0)),
            pl.BlockSpec((H, 2), lambda i: (0, 0)),
            pl.BlockSpec((1, 2), lambda i: (0, 0)),
        ],
        out_specs=pl.BlockSpec((BN, 2), lambda i: (i, 0)),
        out_shape=jax.ShapeDtypeStruct((HALF, 2), jnp.float32),
    )(h2, a2, Wu, bu.reshape(1, H), Wb, bb.reshape(1, 2))


def kernel(x, edge_index, h_msg, Wi, bi, We, be, Wm1, bm1, Wm2, bm2,
           Wn1, bn1, Wn2, bn2, Wu, bu, Wd, bd, Wb, bb):
    h_node = _node_embed(x, Wi, bi)

    src = edge_index[0]
    dst = edge_index[1]
    gs, gd = _sc_gather(h_node, src, dst)

    hnT, h_new, y3 = _edge_compute(gs, gd, h_msg.T, We, be, Wm1, bm1, Wm2, bm2,
                                   Wn1, bn1, Wn2, bn2, Wd, bd)
    h_msg_new = hnT.T
    y_msg = y3.transpose(0, 2, 1).reshape(E, 2)

    aggr = _sc_scatter(h_new, dst)

    y_beliefs = _beliefs(h_node.reshape(HALF, 2, H), aggr.reshape(HALF, 2, H),
                         Wu, bu, Wb, bb)
    return (h_msg_new, y_msg, y_beliefs)


# speedup vs baseline: 1.8370x; 1.2802x over previous
import functools

import jax
import jax.numpy as jnp
from jax import lax
from jax.experimental import pallas as pl
from jax.experimental.pallas import tpu as pltpu
from jax.experimental.pallas import tpu_sc as plsc

N = 100000
E = 1600000
EH = E // 2
H = 32
HALF = N // 2


def _lrelu(v):
    return jnp.where(v > 0, v, 0.01 * v)


def _node_embed(x, Wi, bi):
    BN = 4000

    def body(x_ref, wi_ref, bi_ref, out_ref):
        out_ref[...] = (
            jnp.dot(x_ref[...], wi_ref[...], preferred_element_type=jnp.float32)
            + bi_ref[...]
        )

    return pl.pallas_call(
        body,
        grid=(N // BN,),
        in_specs=[
            pl.BlockSpec((BN, 3), lambda i: (i, 0)),
            pl.BlockSpec((3, H), lambda i: (0, 0)),
            pl.BlockSpec((1, H), lambda i: (0, 0)),
        ],
        out_specs=pl.BlockSpec((BN, H), lambda i: (i, 0)),
        out_shape=jax.ShapeDtypeStruct((N, H), jnp.float32),
    )(x, Wi, bi.reshape(1, H))


def _sc_gather(table, src, dst):
    CH = 640
    NCHUNK = EH // CH
    NW = 32
    PER_W = -(-NCHUNK // NW)
    mesh = plsc.VectorSubcoreMesh(core_axis_name="c", subcore_axis_name="s")

    @functools.partial(
        pl.kernel,
        mesh=mesh,
        compiler_params=pltpu.CompilerParams(use_tc_tiling_on_sc=False),
        out_type=jax.ShapeDtypeStruct((EH, 2 * H), jnp.float32),
        scratch_types=[
            pltpu.VMEM((CH,), jnp.int32),
            pltpu.VMEM((CH,), jnp.int32),
            pltpu.VMEM((CH, H), jnp.float32),
            pltpu.VMEM((CH, H), jnp.float32),
            pltpu.SemaphoreType.DMA,
        ],
    )
    def k(tab, src_hbm, dst_hbm, out, idxs, idxd, rs, rd, sem):
        w = lax.axis_index("s") * 2 + lax.axis_index("c")

        def body(i, carry):
            cid = w + NW * i

            @pl.when(cid < NCHUNK)
            def _():
                base = pl.multiple_of(cid * CH, CH)
                pltpu.sync_copy(src_hbm.at[pl.ds(base, CH)], idxs)
                pltpu.sync_copy(dst_hbm.at[pl.ds(base, CH)], idxd)
                cps = []
                for t in range(CH // 128):
                    cps.append(pltpu.async_copy(
                        tab.at[idxs.at[pl.ds(t * 128, 128)]],
                        rs.at[pl.ds(t * 128, 128), :], sem))
                    cps.append(pltpu.async_copy(
                        tab.at[idxd.at[pl.ds(t * 128, 128)]],
                        rd.at[pl.ds(t * 128, 128), :], sem))
                for cp in cps:
                    cp.wait()
                pltpu.sync_copy(rs, out.at[pl.ds(base, CH), pl.ds(0, H)])
                pltpu.sync_copy(rd, out.at[pl.ds(base, CH), pl.ds(H, H)])

            return carry

        lax.fori_loop(0, PER_W, body, 0)

    return k(table, src, dst)


def _sc_scatter(h_new, dst):
    CH = 640
    NCHUNK = EH // CH
    PER_T = -(-NCHUNK // 16)
    R = 50048
    ZCH = R // 128
    mesh = plsc.VectorSubcoreMesh(core_axis_name="c", subcore_axis_name="s")

    @functools.partial(
        pl.kernel,
        mesh=mesh,
        compiler_params=pltpu.CompilerParams(use_tc_tiling_on_sc=False),
        out_type=jax.ShapeDtypeStruct((N, H), jnp.float32),
        scratch_types=[
            pltpu.VMEM((CH,), jnp.int32),
            pltpu.VMEM((CH // 128, 128), jnp.int32),
            pltpu.VMEM((CH, H), jnp.float32),
            pltpu.VMEM_SHARED((R, H), jnp.float32),
            pltpu.SemaphoreType.DMA,
        ],
    )
    def k(hnew, dref, out, dbuf, ibuf, rbuf, table, sem):
        c = lax.axis_index("c")
        s = lax.axis_index("s")
        nbase = c * HALF
        zeros16 = jnp.zeros((16,), jnp.float32)

        def zrow(r, carry):
            rbuf[r, pl.ds(0, 16)] = zeros16
            rbuf[r, pl.ds(16, 16)] = zeros16
            return carry

        lax.fori_loop(0, 128, zrow, 0)

        def ztab(i, carry):
            z = s + 16 * i

            @pl.when(z < ZCH)
            def _():
                off = pl.multiple_of(z * 128, 128)
                pltpu.sync_copy(rbuf.at[pl.ds(0, 128), :], table.at[pl.ds(off, 128), :])

            return carry

        lax.fori_loop(0, -(-ZCH // 16), ztab, 0)
        plsc.subcore_barrier()

        def chunk(i, carry):
            cid = s + 16 * i

            @pl.when(cid < NCHUNK)
            def _():
                base = pl.multiple_of(cid * CH, CH)
                pltpu.sync_copy(dref.at[pl.ds(base, CH)], dbuf)
                pltpu.sync_copy(hnew.at[pl.ds(base, CH), :], rbuf)
                for j in range(CH // 16):
                    v = dbuf[pl.ds(j * 16, 16)]
                    loc = v - nbase
                    ok = (loc >= 0) & (loc < HALF)
                    loc = jnp.where(ok, loc, HALF)
                    ibuf[j // 8, pl.ds((j % 8) * 16, 16)] = loc
                for t in range(CH // 128):
                    pltpu.sync_copy(
                        rbuf.at[pl.ds(t * 128, 128), :],
                        table.at[ibuf.at[t]],
                        add=True,
                    )

            return carry

        lax.fori_loop(0, PER_T, chunk, 0)
        plsc.subcore_barrier()

        rows_per_t = HALF // 16
        roff = s * rows_per_t
        pltpu.sync_copy(
            table.at[pl.ds(roff, rows_per_t), :],
            out.at[pl.ds(nbase + roff, rows_per_t), :],
        )

    return k(h_new, dst)


def _edge_compute(phase, gxd, hmT, We, be, Wm1, bm1, Wm2, bm2,
                  Wn1, bn1, Wn2, bn2, Wd, bd, hnT_prev, y3_prev):
    EB = 6400
    NBLK = EH // EB
    OFF = phase * NBLK
    f32 = jnp.float32
    dn_lhsT = (((0,), (0,)), ((), ()))

    def body(g_ref, hmT_ref, we_ref, be_ref, wm1_ref, bm1_ref,
             wm2_ref, bm2_ref, wn1_ref, bn1_ref, wn2_ref, bn2_ref,
             wd_ref, bd_ref, *rest):
        hnT_ref, hn_ref, y3_ref = rest[-3:]
        g = g_ref[...]
        wm1 = wm1_ref[...]
        wa = wm1[0:H]
        wb = wm1[H:2 * H]
        wc = wm1[2 * H:3 * H]
        zer = jnp.zeros((H, H), f32)
        wem = jnp.dot(we_ref[...], wc, preferred_element_type=f32)
        bpre = jnp.dot(be_ref[...], wc, preferred_element_type=f32) + bm1_ref[...]
        wsd = jnp.concatenate(
            [jnp.concatenate([wb, wn1_ref[...]], axis=1),
             jnp.concatenate([wa, zer], axis=1)], axis=0)
        wem64 = jnp.concatenate([wem, zer], axis=1)
        b1 = jnp.concatenate([bpre, bn1_ref[...]], axis=1)
        p = _lrelu(
            jnp.dot(g, wsd, preferred_element_type=f32)
            + lax.dot_general(hmT_ref[...], wem64, dn_lhsT,
                              preferred_element_type=f32)
            + b1
        )
        w2 = jnp.concatenate(
            [jnp.concatenate([wm2_ref[...], zer], axis=1),
             jnp.concatenate([zer, wn2_ref[...]], axis=1)], axis=0)
        b2 = jnp.concatenate([bm2_ref[...], bn2_ref[...]], axis=1)
        q = _lrelu(jnp.dot(p, w2, preferred_element_type=f32) + b2)
        hn_ref[...] = q[:, 0:H] + q[:, H:2 * H]
        qT = _lrelu(
            lax.dot_general(w2, p, (((0,), (1,)), ((), ())),
                            preferred_element_type=f32)
            + b2.reshape(2 * H, 1)
        )
        hnT = qT[0:H, :] + qT[H:2 * H, :]
        hnT_ref[...] = hnT
        zT = (lax.dot_general(wd_ref[...], hnT, dn_lhsT,
                              preferred_element_type=f32)
              + bd_ref[...].reshape(2, 1))
        zT = zT - jnp.max(zT, axis=0, keepdims=True)
        ez = jnp.exp(zT)
        yT = ez / jnp.sum(ez, axis=0, keepdims=True)
        y3_ref[...] = yT.reshape(2, EB // 128, 128).transpose(1, 0, 2)

    wspec = pl.BlockSpec((H, H), lambda i: (0, 0))
    bspec = pl.BlockSpec((1, H), lambda i: (0, 0))
    in_specs = [
        pl.BlockSpec((EB, 2 * H), lambda i: (i, 0)),
        pl.BlockSpec((H, EB), lambda i: (0, i + OFF)),
        wspec, bspec,
        pl.BlockSpec((3 * H, H), lambda i: (0, 0)), bspec,
        wspec, bspec,
        wspec, bspec,
        wspec, bspec,
        pl.BlockSpec((H, 2), lambda i: (0, 0)),
        pl.BlockSpec((1, 2), lambda i: (0, 0)),
    ]
    args = [gxd, hmT, We, be.reshape(1, H), Wm1, bm1.reshape(1, H),
            Wm2, bm2.reshape(1, H), Wn1, bn1.reshape(1, H),
            Wn2, bn2.reshape(1, H), Wd, bd.reshape(1, 2)]
    aliases = {}
    if phase == 1:
        in_specs += [pl.BlockSpec(memory_space=pl.ANY),
                     pl.BlockSpec(memory_space=pl.ANY)]
        args += [hnT_prev, y3_prev]
        aliases = {14: 0, 15: 2}
    return pl.pallas_call(
        body,
        grid=(NBLK,),
        in_specs=in_specs,
        out_specs=[
            pl.BlockSpec((H, EB), lambda i: (0, i + OFF)),
            pl.BlockSpec((EB, H), lambda i: (i, 0)),
            pl.BlockSpec((EB // 128, 2, 128), lambda i: (i + OFF, 0, 0)),
        ],
        out_shape=[
            jax.ShapeDtypeStruct((H, E), jnp.float32),
            jax.ShapeDtypeStruct((EH, H), jnp.float32),
            jax.ShapeDtypeStruct((E // 128, 2, 128), jnp.float32),
        ],
        input_output_aliases=aliases,
    )(*args)


def _beliefs(h2, a2a, a2b, Wu, bu, Wb, bb):
    BN = 2000

    def body(h_ref, aa_ref, ab_ref, wu_ref, bu_ref, wb_ref, bb_ref, out_ref):
        f32 = jnp.float32
        hn = h_ref[:, 1, :]
        ag = aa_ref[:, 1, :] + ab_ref[:, 1, :]
        wu = wu_ref[...]
        z = (
            jnp.dot(hn, wu[0:H], preferred_element_type=f32)
            + jnp.dot(ag, wu[H:2 * H], preferred_element_type=f32)
            + bu_ref[...]
        )
        z = _lrelu(z)
        t = jnp.dot(z, wb_ref[...], preferred_element_type=f32) + bb_ref[...]
        t = t - jnp.max(t, axis=-1, keepdims=True)
        et = jnp.exp(t)
        out_ref[...] = et / jnp.sum(et, axis=-1, keepdims=True)

    nspec = pl.BlockSpec((BN, 2, H), lambda i: (i, 0, 0))
    return pl.pallas_call(
        body,
        grid=(HALF // BN,),
        in_specs=[
            nspec, nspec, nspec,
            pl.BlockSpec((2 * H, H), lambda i: (0, 0)),
            pl.BlockSpec((1, H), lambda i: (0, 0)),
            pl.BlockSpec((H, 2), lambda i: (0, 0)),
            pl.BlockSpec((1, 2), lambda i: (0, 0)),
        ],
        out_specs=pl.BlockSpec((BN, 2), lambda i: (i, 0)),
        out_shape=jax.ShapeDtypeStruct((HALF, 2), jnp.float32),
    )(h2, a2a, a2b, Wu, bu.reshape(1, H), Wb, bb.reshape(1, 2))


def kernel(x, edge_index, h_msg, Wi, bi, We, be, Wm1, bm1, Wm2, bm2,
           Wn1, bn1, Wn2, bn2, Wu, bu, Wd, bd, Wb, bb):
    h_node = _node_embed(x, Wi, bi)

    src = edge_index[0]
    dst = edge_index[1]
    hmT = h_msg.T

    gxd1 = _sc_gather(h_node, src[:EH], dst[:EH])
    gxd2 = _sc_gather(h_node, src[EH:], dst[EH:])

    hnT1, h_new1, y31 = _edge_compute(0, gxd1, hmT, We, be, Wm1, bm1, Wm2, bm2,
                                      Wn1, bn1, Wn2, bn2, Wd, bd, None, None)
    aggr1 = _sc_scatter(h_new1, dst[:EH])
    hnT, h_new2, y3 = _edge_compute(1, gxd2, hmT, We, be, Wm1, bm1, Wm2, bm2,
                                    Wn1, bn1, Wn2, bn2, Wd, bd, hnT1, y31)
    aggr2 = _sc_scatter(h_new2, dst[EH:])

    h_msg_new = hnT.T
    y_msg = y3.transpose(0, 2, 1).reshape(E, 2)

    y_beliefs = _beliefs(h_node.reshape(HALF, 2, H), aggr1.reshape(HALF, 2, H),
                         aggr2.reshape(HALF, 2, H), Wu, bu, Wb, bb)
    return (h_msg_new, y_msg, y_beliefs)
